# speedup vs baseline: 2.3620x; 2.3620x over previous
import jax
import jax.numpy as jnp
from jax.experimental import pallas as pl
from jax.experimental.pallas import tpu as pltpu

_INV_T = 5.0
_QB = 1024


def _norm_rows(x):
    return x * jax.lax.rsqrt(jnp.maximum(jnp.sum(x * x, axis=1, keepdims=True),
                                         1e-24))


def _corr_body(d1_ref, d2_ref, qn_ref, pos_ref):
    xn = _norm_rows(d1_ref[0])
    yn = _norm_rows(d2_ref[0])
    sim = jax.lax.dot_general(xn, yn, (((1,), (1,)), ((), ())),
                              preferred_element_type=jnp.float32)
    qn_ref[0] = xn
    pos_ref[0] = jnp.max(sim, axis=1, keepdims=True) * _INV_T


def _neg_body(qn_ref, queue_ref, neg_ref, part_ref):
    qbn = _norm_rows(queue_ref[...])
    neg = jax.lax.dot_general(qn_ref[...], qbn, (((1,), (1,)), ((), ())),
                              preferred_element_type=jnp.float32) * _INV_T
    neg_ref[...] = neg
    part_ref[0] = jnp.sum(jnp.exp(neg), axis=1, keepdims=True)


def _loss_body(pos_ref, part_ref, loss_ref):
    pos = pos_ref[...]
    tot = jnp.sum(part_ref[...], axis=0, keepdims=True) + jnp.exp(pos)
    per_row = jnp.log(tot) - pos
    loss_ref[...] = jnp.sum(per_row, axis=1, keepdims=True) * (1.0 / pos.shape[1])


@jax.jit
def _impl(d1, d2, queue):
    B, H, W, D = d1.shape
    N = H * W
    R = B * N
    Q = queue.shape[0]
    nq = Q // _QB

    d1r = d1.reshape(B, N, D)
    d2r = d2.reshape(B, N, D)

    qn3, pos3 = pl.pallas_call(
        _corr_body,
        grid=(B,),
        in_specs=[pl.BlockSpec((1, N, D), lambda b: (b, 0, 0)),
                  pl.BlockSpec((1, N, D), lambda b: (b, 0, 0))],
        out_specs=[pl.BlockSpec((1, N, D), lambda b: (b, 0, 0)),
                   pl.BlockSpec((1, N, 1), lambda b: (b, 0, 0))],
        out_shape=[jax.ShapeDtypeStruct((B, N, D), jnp.float32),
                   jax.ShapeDtypeStruct((B, N, 1), jnp.float32)],
        compiler_params=pltpu.CompilerParams(
            dimension_semantics=("parallel",)),
        name="corr_pos",
    )(d1r, d2r)

    qn = qn3.reshape(R, D)
    neg, parts = pl.pallas_call(
        _neg_body,
        grid=(nq,),
        in_specs=[pl.BlockSpec((R, D), lambda j: (0, 0)),
                  pl.BlockSpec((_QB, D), lambda j: (j, 0))],
        out_specs=[pl.BlockSpec((R, _QB), lambda j: (0, j)),
                   pl.BlockSpec((1, R, 1), lambda j: (j, 0, 0))],
        out_shape=[jax.ShapeDtypeStruct((R, Q), jnp.float32),
                   jax.ShapeDtypeStruct((nq, R, 1), jnp.float32)],
        compiler_params=pltpu.CompilerParams(
            dimension_semantics=("parallel",),
            vmem_limit_bytes=50 * 1024 * 1024),
        name="neg_queue",
    )(qn, queue)

    loss = pl.pallas_call(
        _loss_body,
        out_shape=jax.ShapeDtypeStruct((1, 1), jnp.float32),
        name="loss_combine",
    )(pos3.reshape(1, R), parts.reshape(nq, R))

    return loss[0, 0], pos3.reshape(B, N), neg


def kernel(dense_features_1, dense_features_2, backbone_features_1,
           backbone_features_2, queue):
    del backbone_features_1, backbone_features_2
    return _impl(dense_features_1, dense_features_2, queue)


# speedup vs baseline: 2.8925x; 1.2246x over previous
import jax
import jax.numpy as jnp
from jax.experimental import pallas as pl
from jax.experimental.pallas import tpu as pltpu

_INV_T = 5.0
_QB = 1024
_BB = 4


def _norm_rows(x, axis):
    return x * jax.lax.rsqrt(
        jnp.maximum(jnp.sum(x * x, axis=axis, keepdims=True), 1e-24))


def _corr_body(d1_ref, d2_ref, qn_ref, pos_ref):
    xn = _norm_rows(d1_ref[...], 2)
    yn = _norm_rows(d2_ref[...], 2)
    qn_ref[...] = xn
    for i in range(_BB):
        sim = jax.lax.dot_general(xn[i], yn[i], (((1,), (1,)), ((), ())),
                                  preferred_element_type=jnp.float32)
        pos_ref[i] = jnp.max(sim, axis=1, keepdims=True) * _INV_T


def _neg_body(qn_ref, queue_ref, neg_ref, part_ref):
    j = pl.program_id(1)
    qbn = _norm_rows(queue_ref[...], 1)
    neg = jax.lax.dot_general(qn_ref[...], qbn, (((1,), (1,)), ((), ())),
                              preferred_element_type=jnp.float32) * _INV_T
    neg_ref[...] = neg
    ex = jnp.exp(neg)
    lp = ex[:, 0:128]
    for k in range(1, _QB // 128):
        lp = lp + ex[:, k * 128:(k + 1) * 128]

    @pl.when(j == 0)
    def _():
        part_ref[0] = lp

    @pl.when(j > 0)
    def _():
        part_ref[0] += lp


def _loss_body(pos_ref, part_ref, loss_ref):
    pos = pos_ref[...]
    t = part_ref[0] + part_ref[1]
    tot = jnp.sum(t, axis=1, keepdims=True) + jnp.exp(pos)
    per_row = jnp.log(tot) - pos
    loss_ref[...] = jnp.sum(per_row, axis=0, keepdims=True) * (1.0 / pos.shape[0])


@jax.jit
def _impl(d1, d2, queue):
    B, H, W, D = d1.shape
    N = H * W
    R = B * N
    Q = queue.shape[0]
    nq2 = Q // _QB // 2

    d1r = d1.reshape(B, N, D)
    d2r = d2.reshape(B, N, D)

    qn3, pos3 = pl.pallas_call(
        _corr_body,
        grid=(B // _BB,),
        in_specs=[pl.BlockSpec((_BB, N, D), lambda b: (b, 0, 0)),
                  pl.BlockSpec((_BB, N, D), lambda b: (b, 0, 0))],
        out_specs=[pl.BlockSpec((_BB, N, D), lambda b: (b, 0, 0)),
                   pl.BlockSpec((_BB, N, 1), lambda b: (b, 0, 0))],
        out_shape=[jax.ShapeDtypeStruct((B, N, D), jnp.float32),
                   jax.ShapeDtypeStruct((B, N, 1), jnp.float32)],
        compiler_params=pltpu.CompilerParams(
            dimension_semantics=("parallel",)),
        name="corr_pos",
    )(d1r, d2r)

    qn = qn3.reshape(R, D)
    neg, parts = pl.pallas_call(
        _neg_body,
        grid=(2, nq2),
        in_specs=[pl.BlockSpec((R, D), lambda c, j: (0, 0)),
                  pl.BlockSpec((_QB, D), lambda c, j: (c * nq2 + j, 0))],
        out_specs=[pl.BlockSpec((R, _QB), lambda c, j: (0, c * nq2 + j)),
                   pl.BlockSpec((1, R, 128), lambda c, j: (c, 0, 0))],
        out_shape=[jax.ShapeDtypeStruct((R, Q), jnp.float32),
                   jax.ShapeDtypeStruct((2, R, 128), jnp.float32)],
        compiler_params=pltpu.CompilerParams(
            dimension_semantics=("parallel", "arbitrary"),
            vmem_limit_bytes=50 * 1024 * 1024),
        name="neg_queue",
    )(qn, queue)

    loss = pl.pallas_call(
        _loss_body,
        out_shape=jax.ShapeDtypeStruct((1, 1), jnp.float32),
        name="loss_combine",
    )(pos3.reshape(R, 1), parts)

    return loss[0, 0], pos3.reshape(B, N), neg


def kernel(dense_features_1, dense_features_2, backbone_features_1,
           backbone_features_2, queue):
    del backbone_features_1, backbone_features_2
    return _impl(dense_features_1, dense_features_2, queue)
